# baseline (speedup 1.0000x reference)
import jax
import jax.numpy as jnp
from jax.experimental import pallas as pl
from jax.experimental.pallas import tpu as pltpu

_C = 8
_THR = 0.8
_BLK = 1024


def _tc_body(lab_ref, w_ref, bias_ref, x_ref, feats_ref, scores_ref, acc_ref):
    b = pl.program_id(0)
    i = pl.program_id(1)
    ni = pl.num_programs(1)

    x = x_ref[0].astype(jnp.bfloat16)
    w = w_ref[...].astype(jnp.bfloat16)
    proj = jax.lax.dot_general(
        x, w, (((1,), (0,)), ((), ())),
        preferred_element_type=jnp.float32)
    ones = jnp.ones((_BLK, 1), jnp.float32)
    proj_aug = jnp.concatenate([proj, ones], axis=1)

    lab = lab_ref[pl.ds(b, 1), pl.ds(i * _BLK, _BLK)]
    cid = jax.lax.broadcasted_iota(jnp.int32, (_C, _BLK), 0)
    oh = (jnp.broadcast_to(lab, (_C, _BLK)) == cid).astype(jnp.bfloat16)

    seg = jax.lax.dot_general(
        oh, proj_aug.astype(jnp.bfloat16), (((1,), (0,)), ((), ())),
        preferred_element_type=jnp.float32)

    @pl.when(i == 0)
    def _init():
        acc_ref[...] = seg

    @pl.when(i != 0)
    def _accum():
        acc_ref[...] = acc_ref[...] + seg

    @pl.when(i == ni - 1)
    def _epilogue():
        a = acc_ref[...]
        cnt = jnp.maximum(a[:, 2:3], 1.0)
        l = a[:, 0:2] / cnt + bias_ref[...]
        m = jnp.max(l, axis=1, keepdims=True)
        e0 = jnp.exp(l[:, 0:1] - m)
        e1 = jnp.exp(l[:, 1:2] - m)
        sc = e1 / (e0 + e1)
        row = jax.lax.broadcasted_iota(jnp.int32, (_C, 1), 0)
        mx = jnp.max(sc)
        mn = jnp.min(sc)
        idx_max = jnp.min(jnp.where(sc == mx, row, _C))
        idx_min = jnp.min(jnp.where(sc == mn, row, _C))
        sel = jnp.where(mx < _THR, idx_min, idx_max)
        selh = (row == sel).astype(jnp.float32)
        f0 = jnp.sum(l[:, 0:1] * selh)
        f1 = jnp.sum(l[:, 1:2] * selh)
        feats_ref[pl.ds(b, 1), :] = jnp.concatenate(
            [f0.reshape(1, 1), f1.reshape(1, 1)], axis=1)
        eye = (jax.lax.broadcasted_iota(jnp.int32, (_C, _C), 0)
               == jax.lax.broadcasted_iota(jnp.int32, (_C, _C), 1)
               ).astype(jnp.float32)
        sc_row = jnp.sum(sc * eye, axis=0, keepdims=True)
        scores_ref[pl.ds(b, 1), :] = sc_row


def kernel(inst_feat, cluster_labels, W_head, b_head):
    B, N, D = inst_feat.shape
    ncls = W_head.shape[1]
    bias = b_head.reshape(1, ncls)
    grid = (B, N // _BLK)
    feats, scores = pl.pallas_call(
        _tc_body,
        grid=grid,
        in_specs=[
            pl.BlockSpec((B, N), lambda b, i: (0, 0)),
            pl.BlockSpec((D, ncls), lambda b, i: (0, 0)),
            pl.BlockSpec((1, ncls), lambda b, i: (0, 0)),
            pl.BlockSpec((1, _BLK, D), lambda b, i: (b, i, 0)),
        ],
        out_specs=[
            pl.BlockSpec((B, ncls), lambda b, i: (0, 0)),
            pl.BlockSpec((B, _C), lambda b, i: (0, 0)),
        ],
        out_shape=[
            jax.ShapeDtypeStruct((B, ncls), jnp.float32),
            jax.ShapeDtypeStruct((B, _C), jnp.float32),
        ],
        scratch_shapes=[pltpu.VMEM((_C, 3), jnp.float32)],
    )(cluster_labels, W_head, bias, inst_feat)
    return feats, scores


# speedup vs baseline: 1.2211x; 1.2211x over previous
import jax
import jax.numpy as jnp
from jax.experimental import pallas as pl
from jax.experimental.pallas import tpu as pltpu

_C = 8
_THR = 0.8
_BLK = 2048


def _tc_body(lab_ref, w_ref, bias_ref, x_ref, feats_ref, scores_ref, acc_ref):
    b = pl.program_id(0)
    i = pl.program_id(1)
    ni = pl.num_programs(1)

    x = x_ref[0].astype(jnp.bfloat16)
    w = w_ref[...].astype(jnp.bfloat16)
    proj = jax.lax.dot_general(
        x, w, (((1,), (0,)), ((), ())),
        preferred_element_type=jnp.float32)
    ones = jnp.ones((_BLK, 1), jnp.float32)
    proj_aug = jnp.concatenate([proj, ones], axis=1)

    lab = lab_ref[pl.ds(b, 1), pl.ds(i * _BLK, _BLK)]
    cid = jax.lax.broadcasted_iota(jnp.int32, (_C, _BLK), 0)
    oh = (jnp.broadcast_to(lab, (_C, _BLK)) == cid).astype(jnp.bfloat16)

    seg = jax.lax.dot_general(
        oh, proj_aug.astype(jnp.bfloat16), (((1,), (0,)), ((), ())),
        preferred_element_type=jnp.float32)

    @pl.when(i == 0)
    def _init():
        acc_ref[...] = seg

    @pl.when(i != 0)
    def _accum():
        acc_ref[...] = acc_ref[...] + seg

    @pl.when(i == ni - 1)
    def _epilogue():
        a = acc_ref[...]
        cnt = jnp.maximum(a[:, 2:3], 1.0)
        l = a[:, 0:2] / cnt + bias_ref[...]
        m = jnp.max(l, axis=1, keepdims=True)
        e0 = jnp.exp(l[:, 0:1] - m)
        e1 = jnp.exp(l[:, 1:2] - m)
        sc = e1 / (e0 + e1)
        row = jax.lax.broadcasted_iota(jnp.int32, (_C, 1), 0)
        mx = jnp.max(sc)
        mn = jnp.min(sc)
        idx_max = jnp.min(jnp.where(sc == mx, row, _C))
        idx_min = jnp.min(jnp.where(sc == mn, row, _C))
        sel = jnp.where(mx < _THR, idx_min, idx_max)
        selh = (row == sel).astype(jnp.float32)
        f0 = jnp.sum(l[:, 0:1] * selh)
        f1 = jnp.sum(l[:, 1:2] * selh)
        feats_ref[pl.ds(b, 1), :] = jnp.concatenate(
            [f0.reshape(1, 1), f1.reshape(1, 1)], axis=1)
        eye = (jax.lax.broadcasted_iota(jnp.int32, (_C, _C), 0)
               == jax.lax.broadcasted_iota(jnp.int32, (_C, _C), 1)
               ).astype(jnp.float32)
        sc_row = jnp.sum(sc * eye, axis=0, keepdims=True)
        scores_ref[pl.ds(b, 1), :] = sc_row


def kernel(inst_feat, cluster_labels, W_head, b_head):
    B, N, D = inst_feat.shape
    ncls = W_head.shape[1]
    bias = b_head.reshape(1, ncls)
    grid = (B, N // _BLK)
    feats, scores = pl.pallas_call(
        _tc_body,
        grid=grid,
        in_specs=[
            pl.BlockSpec((B, N), lambda b, i: (0, 0)),
            pl.BlockSpec((D, ncls), lambda b, i: (0, 0)),
            pl.BlockSpec((1, ncls), lambda b, i: (0, 0)),
            pl.BlockSpec((1, _BLK, D), lambda b, i: (b, i, 0)),
        ],
        out_specs=[
            pl.BlockSpec((B, ncls), lambda b, i: (0, 0)),
            pl.BlockSpec((B, _C), lambda b, i: (0, 0)),
        ],
        out_shape=[
            jax.ShapeDtypeStruct((B, ncls), jnp.float32),
            jax.ShapeDtypeStruct((B, _C), jnp.float32),
        ],
        scratch_shapes=[pltpu.VMEM((_C, 3), jnp.float32)],
    )(cluster_labels, W_head, bias, inst_feat)
    return feats, scores


# speedup vs baseline: 1.4398x; 1.1791x over previous
import jax
import jax.numpy as jnp
from jax.experimental import pallas as pl
from jax.experimental.pallas import tpu as pltpu

_C = 8
_THR = 0.8
_BLK = 4096


def _tc_body(lab_ref, w_ref, bias_ref, x_ref, feats_ref, scores_ref, acc_ref):
    b = pl.program_id(0)
    i = pl.program_id(1)
    ni = pl.num_programs(1)

    x = x_ref[0].astype(jnp.bfloat16)
    w = w_ref[...].astype(jnp.bfloat16)
    proj = jax.lax.dot_general(
        x, w, (((1,), (0,)), ((), ())),
        preferred_element_type=jnp.float32)
    ones = jnp.ones((_BLK, 1), jnp.float32)
    proj_aug = jnp.concatenate([proj, ones], axis=1)

    lab = lab_ref[pl.ds(b, 1), pl.ds(i * _BLK, _BLK)]
    cid = jax.lax.broadcasted_iota(jnp.int32, (_C, _BLK), 0)
    oh = (jnp.broadcast_to(lab, (_C, _BLK)) == cid).astype(jnp.bfloat16)

    seg = jax.lax.dot_general(
        oh, proj_aug.astype(jnp.bfloat16), (((1,), (0,)), ((), ())),
        preferred_element_type=jnp.float32)

    @pl.when(i == 0)
    def _init():
        acc_ref[...] = seg

    @pl.when(i != 0)
    def _accum():
        acc_ref[...] = acc_ref[...] + seg

    @pl.when(i == ni - 1)
    def _epilogue():
        a = acc_ref[...]
        cnt = jnp.maximum(a[:, 2:3], 1.0)
        l = a[:, 0:2] / cnt + bias_ref[...]
        m = jnp.max(l, axis=1, keepdims=True)
        e0 = jnp.exp(l[:, 0:1] - m)
        e1 = jnp.exp(l[:, 1:2] - m)
        sc = e1 / (e0 + e1)
        row = jax.lax.broadcasted_iota(jnp.int32, (_C, 1), 0)
        mx = jnp.max(sc)
        mn = jnp.min(sc)
        idx_max = jnp.min(jnp.where(sc == mx, row, _C))
        idx_min = jnp.min(jnp.where(sc == mn, row, _C))
        sel = jnp.where(mx < _THR, idx_min, idx_max)
        selh = (row == sel).astype(jnp.float32)
        f0 = jnp.sum(l[:, 0:1] * selh)
        f1 = jnp.sum(l[:, 1:2] * selh)
        feats_ref[pl.ds(b, 1), :] = jnp.concatenate(
            [f0.reshape(1, 1), f1.reshape(1, 1)], axis=1)
        eye = (jax.lax.broadcasted_iota(jnp.int32, (_C, _C), 0)
               == jax.lax.broadcasted_iota(jnp.int32, (_C, _C), 1)
               ).astype(jnp.float32)
        sc_row = jnp.sum(sc * eye, axis=0, keepdims=True)
        scores_ref[pl.ds(b, 1), :] = sc_row


def kernel(inst_feat, cluster_labels, W_head, b_head):
    B, N, D = inst_feat.shape
    ncls = W_head.shape[1]
    bias = b_head.reshape(1, ncls)
    grid = (B, N // _BLK)
    feats, scores = pl.pallas_call(
        _tc_body,
        grid=grid,
        in_specs=[
            pl.BlockSpec((B, N), lambda b, i: (0, 0)),
            pl.BlockSpec((D, ncls), lambda b, i: (0, 0)),
            pl.BlockSpec((1, ncls), lambda b, i: (0, 0)),
            pl.BlockSpec((1, _BLK, D), lambda b, i: (b, i, 0)),
        ],
        out_specs=[
            pl.BlockSpec((B, ncls), lambda b, i: (0, 0)),
            pl.BlockSpec((B, _C), lambda b, i: (0, 0)),
        ],
        out_shape=[
            jax.ShapeDtypeStruct((B, ncls), jnp.float32),
            jax.ShapeDtypeStruct((B, _C), jnp.float32),
        ],
        scratch_shapes=[pltpu.VMEM((_C, 3), jnp.float32)],
    )(cluster_labels, W_head, bias, inst_feat)
    return feats, scores


# speedup vs baseline: 1.4623x; 1.0156x over previous
import jax
import jax.numpy as jnp
from jax.experimental import pallas as pl
from jax.experimental.pallas import tpu as pltpu

_C = 8
_THR = 0.8


def _tc_body(lab_ref, w_ref, bias_ref, x_ref, feats_ref, scores_ref,
             s0_ref, s1_ref, cn_ref):
    b = pl.program_id(0)
    nb = pl.num_programs(0)
    n = x_ref.shape[1]

    x = x_ref[0].astype(jnp.bfloat16)
    w = w_ref[...].astype(jnp.bfloat16)
    proj = jax.lax.dot_general(
        x, w, (((1,), (0,)), ((), ())),
        preferred_element_type=jnp.float32)
    ones = jnp.ones((n, 1), jnp.float32)
    proj_aug = jnp.concatenate([proj, ones], axis=1)

    lab = lab_ref[pl.ds(b, 1), :]
    cid = jax.lax.broadcasted_iota(jnp.int32, (_C, n), 0)
    oh = (jnp.broadcast_to(lab, (_C, n)) == cid).astype(jnp.bfloat16)

    seg = jax.lax.dot_general(
        oh, proj_aug.astype(jnp.bfloat16), (((1,), (0,)), ((), ())),
        preferred_element_type=jnp.float32)

    eye = (jax.lax.broadcasted_iota(jnp.int32, (_C, _C), 0)
           == jax.lax.broadcasted_iota(jnp.int32, (_C, _C), 1)
           ).astype(jnp.float32)
    s0_ref[pl.ds(b, 1), :] = jnp.sum(seg[:, 0:1] * eye, axis=0, keepdims=True)
    s1_ref[pl.ds(b, 1), :] = jnp.sum(seg[:, 1:2] * eye, axis=0, keepdims=True)
    cn_ref[pl.ds(b, 1), :] = jnp.sum(seg[:, 2:3] * eye, axis=0, keepdims=True)

    @pl.when(b == nb - 1)
    def _epilogue():
        nb_ = feats_ref.shape[0]
        cnt = jnp.maximum(cn_ref[...], 1.0)
        l0 = s0_ref[...] / cnt + bias_ref[0, 0]
        l1 = s1_ref[...] / cnt + bias_ref[0, 1]
        m = jnp.maximum(l0, l1)
        e0 = jnp.exp(l0 - m)
        e1 = jnp.exp(l1 - m)
        sc = e1 / (e0 + e1)
        lane = jax.lax.broadcasted_iota(jnp.int32, (nb_, _C), 1)
        mx = jnp.max(sc, axis=1, keepdims=True)
        mn = jnp.min(sc, axis=1, keepdims=True)
        idx_max = jnp.min(jnp.where(sc == mx, lane, _C), axis=1, keepdims=True)
        idx_min = jnp.min(jnp.where(sc == mn, lane, _C), axis=1, keepdims=True)
        sel = jnp.where(mx < _THR, idx_min, idx_max)
        selh = (lane == sel).astype(jnp.float32)
        f0 = jnp.sum(l0 * selh, axis=1, keepdims=True)
        f1 = jnp.sum(l1 * selh, axis=1, keepdims=True)
        feats_ref[...] = jnp.concatenate([f0, f1], axis=1)
        scores_ref[...] = sc


def kernel(inst_feat, cluster_labels, W_head, b_head):
    B, N, D = inst_feat.shape
    ncls = W_head.shape[1]
    bias = b_head.reshape(1, ncls)
    feats, scores = pl.pallas_call(
        _tc_body,
        grid=(B,),
        in_specs=[
            pl.BlockSpec((B, N), lambda b: (0, 0)),
            pl.BlockSpec((D, ncls), lambda b: (0, 0)),
            pl.BlockSpec((1, ncls), lambda b: (0, 0)),
            pl.BlockSpec((1, N, D), lambda b: (b, 0, 0)),
        ],
        out_specs=[
            pl.BlockSpec((B, ncls), lambda b: (0, 0)),
            pl.BlockSpec((B, _C), lambda b: (0, 0)),
        ],
        out_shape=[
            jax.ShapeDtypeStruct((B, ncls), jnp.float32),
            jax.ShapeDtypeStruct((B, _C), jnp.float32),
        ],
        scratch_shapes=[
            pltpu.VMEM((B, _C), jnp.float32),
            pltpu.VMEM((B, _C), jnp.float32),
            pltpu.VMEM((B, _C), jnp.float32),
        ],
    )(cluster_labels, W_head, bias, inst_feat)
    return feats, scores
